# Initial kernel scaffold; baseline (speedup 1.0000x reference)
import jax
import jax.numpy as jnp
from jax.experimental import pallas as pl

N = 10000
FANOUT = 32
D_FEAT = 128
OUT_DIM = 128
BLOCK_ROWS = 500


def _body(src_ref, dst_ref, w1_ref, w2_ref, b_ref, out_ref):
    agg = jnp.mean(dst_ref[...], axis=1)
    out_ref[...] = (
        jnp.dot(src_ref[...], w1_ref[...], preferred_element_type=jnp.float32)
        + jnp.dot(agg, w2_ref[...], preferred_element_type=jnp.float32)
        + b_ref[...]
    )


def kernel(src_feature, dst_feature, W, b):
    w1 = W[:D_FEAT]
    w2 = W[D_FEAT:]
    b2d = b.reshape(1, OUT_DIM)
    grid = (N // BLOCK_ROWS,)
    return pl.pallas_call(
        _body,
        grid=grid,
        in_specs=[
            pl.BlockSpec((BLOCK_ROWS, D_FEAT), lambda i: (i, 0)),
            pl.BlockSpec((BLOCK_ROWS, FANOUT, D_FEAT), lambda i: (i, 0, 0)),
            pl.BlockSpec((D_FEAT, OUT_DIM), lambda i: (0, 0)),
            pl.BlockSpec((D_FEAT, OUT_DIM), lambda i: (0, 0)),
            pl.BlockSpec((1, OUT_DIM), lambda i: (0, 0)),
        ],
        out_specs=pl.BlockSpec((BLOCK_ROWS, OUT_DIM), lambda i: (i, 0)),
        out_shape=jax.ShapeDtypeStruct((N, OUT_DIM), jnp.float32),
    )(src_feature, dst_feature, w1, w2, b2d)


# speedup vs baseline: 1.0980x; 1.0980x over previous
import jax
import jax.numpy as jnp
from jax.experimental import pallas as pl

N = 10000
FANOUT = 32
D_FEAT = 128
OUT_DIM = 128
BLOCK_ROWS = 400


def _body(src_ref, dst_ref, w1_ref, w2_ref, b_ref, out_ref):
    agg = jnp.mean(dst_ref[...], axis=1)
    out_ref[...] = (
        jnp.dot(src_ref[...], w1_ref[...], preferred_element_type=jnp.float32)
        + jnp.dot(agg, w2_ref[...], preferred_element_type=jnp.float32)
        + b_ref[0:1, :]
    )


def kernel(src_feature, dst_feature, W, b):
    w1 = W[:D_FEAT]
    w2 = W[D_FEAT:]
    b2d = jnp.broadcast_to(b.reshape(1, OUT_DIM), (8, OUT_DIM))
    grid = (N // BLOCK_ROWS,)
    return pl.pallas_call(
        _body,
        grid=grid,
        in_specs=[
            pl.BlockSpec((BLOCK_ROWS, D_FEAT), lambda i: (i, 0)),
            pl.BlockSpec((BLOCK_ROWS, FANOUT, D_FEAT), lambda i: (i, 0, 0)),
            pl.BlockSpec((D_FEAT, OUT_DIM), lambda i: (0, 0)),
            pl.BlockSpec((D_FEAT, OUT_DIM), lambda i: (0, 0)),
            pl.BlockSpec((8, OUT_DIM), lambda i: (0, 0)),
        ],
        out_specs=pl.BlockSpec((BLOCK_ROWS, OUT_DIM), lambda i: (i, 0)),
        out_shape=jax.ShapeDtypeStruct((N, OUT_DIM), jnp.float32),
    )(src_feature, dst_feature, w1, w2, b2d)
